# baseline (speedup 1.0000x reference)
import functools

import jax
import jax.numpy as jnp
from jax import lax
from jax.experimental import pallas as pl
from jax.experimental.pallas import tpu as pltpu
from jax.experimental.pallas import tpu_sc as plsc

_B = 16384
_F = 26
_VOCAB = 100000
_DIM = 32
_LANE = 128
_RPL = _LANE // _DIM

_NC = 2
_NS = 16
_NW = _NC * _NS
_ROWS_PER_W = _B // _NW
_CB = 4
_REAL_IDX = _CB * _F
_CHUNK_IDX = 112
_NCHUNKS = _ROWS_PER_W // _CB
_NBUF = 4


def _sc_body(line_hbm, colb_hbm, table_hbm, out_hbm,
             line_v, colb_v, bufs, out_v, sems):
    wid = lax.axis_index("s") * _NC + lax.axis_index("c")

    pltpu.sync_copy(line_hbm.at[wid], line_v)
    pltpu.sync_copy(colb_hbm.at[wid], colb_v)

    iota = lax.broadcasted_iota(jnp.int32, (16,), 0)

    def start(chunk, k):
        pltpu.async_copy(table_hbm.at[line_v.at[chunk]], bufs[k], sems[k])

    def wait(chunk, k):
        pltpu.make_async_copy(
            table_hbm.at[line_v.at[chunk]], bufs[k], sems[k]).wait()

    def accum(chunk, k):
        buf = bufs[k]
        chunkv = jnp.full((16,), chunk, jnp.int32)
        for lb in range(_CB):
            base = lb * _F
            acc0 = None
            acc1 = None
            for f in range(_F):
                j = base + f
                jv = jnp.full((16,), j, jnp.int32)
                cb = plsc.load_gather(colb_v, [chunkv, jv])
                col = cb + iota
                g0 = plsc.load_gather(buf, [jv, col])
                g1 = plsc.load_gather(buf, [jv, col + 16])
                acc0 = g0 if acc0 is None else acc0 + g0
                acc1 = g1 if acc1 is None else acc1 + g1
            row = chunk * _CB + lb
            out_v[row, pl.ds(0, 16)] = acc0
            out_v[row, pl.ds(16, 16)] = acc1

    for k in range(_NBUF):
        start(k, k)

    @pl.loop(0, _NCHUNKS - _NBUF, step=_NBUF)
    def _(c):
        for k in range(_NBUF):
            chunk = c + k
            wait(chunk, k)
            accum(chunk, k)
            start(chunk + _NBUF, k)

    for k in range(_NBUF):
        chunk = _NCHUNKS - _NBUF + k
        wait(chunk, k)
        accum(chunk, k)

    pltpu.sync_copy(out_v, out_hbm.at[pl.ds(wid * _ROWS_PER_W, _ROWS_PER_W)])


@jax.jit
def _multi_embed(line_idx, colb, table_lines):
    mesh = plsc.VectorSubcoreMesh(
        core_axis_name="c", subcore_axis_name="s",
        num_cores=_NC, num_subcores=_NS)
    run = pl.kernel(
        _sc_body,
        out_type=jax.ShapeDtypeStruct((_B, _DIM), jnp.float32),
        mesh=mesh,
        scratch_types=[
            pltpu.VMEM((_NCHUNKS, _CHUNK_IDX), jnp.int32),
            pltpu.VMEM((_NCHUNKS, _CHUNK_IDX), jnp.int32),
            [pltpu.VMEM((_CHUNK_IDX, _LANE), jnp.float32)
             for _ in range(_NBUF)],
            pltpu.VMEM((_ROWS_PER_W, _DIM), jnp.float32),
            [pltpu.SemaphoreType.DMA for _ in range(_NBUF)],
        ],
        compiler_params=pltpu.CompilerParams(
            use_tc_tiling_on_sc=False, needs_layout_passes=False),
    )
    return run(line_idx, colb, table_lines)


def kernel(inputs, tables):
    table_lines = tables.reshape(_F * _VOCAB // _RPL, _LANE)
    offs = (jnp.arange(_F, dtype=jnp.int32) * _VOCAB)[None, :]
    idx = inputs.astype(jnp.int32) + offs
    line_idx = (idx // _RPL).reshape(_NW, _NCHUNKS, _REAL_IDX)
    colbase = ((idx % _RPL) * _DIM).reshape(_NW, _NCHUNKS, _REAL_IDX)
    pad = ((0, 0), (0, 0), (0, _CHUNK_IDX - _REAL_IDX))
    line_idx = jnp.pad(line_idx, pad)
    colbase = jnp.pad(colbase, pad)
    return _multi_embed(line_idx, colbase, table_lines)


# speedup vs baseline: 6.3621x; 6.3621x over previous
import jax
import jax.numpy as jnp
from jax import lax
from jax.experimental import pallas as pl
from jax.experimental.pallas import tpu as pltpu
from jax.experimental.pallas import tpu_sc as plsc

_B = 16384
_F = 26
_VOCAB = 100000
_DIM = 32

_NC = 2
_NS = 16
_NW = _NC * _NS

_IH = _B // 2
_GRP = 4


_VMAIN = 99968
_VPAD = 100096


def _sc_body(sidx_hbm, idx_hbm, table_hbm, tail_hbm, out_hbm,
             fbuf, buf, idx_v, acc, sem, isem):
    d = lax.axis_index("s") * _NC + lax.axis_index("c")

    pltpu.sync_copy(sidx_hbm.at[pl.ds(d * (_F * 8), _F * 8)], fbuf)
    zero = jnp.zeros((16,), jnp.int32)

    for f in range(_F):
        src = table_hbm.at[:, pl.ds(0, _VMAIN)].at[fbuf.at[pl.ds(f * 8, 1)]]
        dst = buf.at[:, pl.ds(0, _VMAIN)]
        pltpu.async_copy(src, dst, sem)
        pltpu.sync_copy(tail_hbm.at[pl.ds(f * (_DIM * 32) + d * 32, 32)],
                        buf.at[0, pl.ds(_VMAIN, 32)])
        pltpu.async_copy(idx_hbm.at[pl.ds(f * _B, _IH)], idx_v, isem)
        pltpu.make_async_copy(src, dst, sem).wait()

        for h in range(2):
            pltpu.make_async_copy(
                idx_hbm.at[pl.ds(f * _B + h * _IH, _IH)], idx_v,
                isem).wait()

            @pl.loop(0, _IH // (16 * _GRP))
            def _(c):
                base = c * (16 * _GRP)
                for k in range(_GRP):
                    ipos = base + k * 16
                    opos = h * _IH + ipos
                    iv = idx_v[pl.ds(ipos, 16)]
                    g = plsc.load_gather(buf, [zero, iv])
                    if f == 0:
                        acc[pl.ds(opos, 16)] = g
                    else:
                        acc[pl.ds(opos, 16)] = acc[pl.ds(opos, 16)] + g

            if h == 0:
                pltpu.async_copy(
                    idx_hbm.at[pl.ds(f * _B + _IH, _IH)], idx_v, isem)

    pltpu.sync_copy(acc, out_hbm.at[pl.ds(d * _B, _B)])


@jax.jit
def _dim_gather_sum(sidx, idx_flat, table2d, tails):
    mesh = plsc.VectorSubcoreMesh(
        core_axis_name="c", subcore_axis_name="s",
        num_cores=_NC, num_subcores=_NS)
    run = pl.kernel(
        _sc_body,
        out_type=jax.ShapeDtypeStruct((_DIM * _B,), jnp.float32),
        mesh=mesh,
        scratch_types=[
            pltpu.VMEM((_F * 8,), jnp.int32),
            pltpu.VMEM((1, _VPAD), jnp.float32),
            pltpu.VMEM((_IH,), jnp.int32),
            pltpu.VMEM((_B,), jnp.float32),
            pltpu.SemaphoreType.DMA,
            pltpu.SemaphoreType.DMA,
        ],
        compiler_params=pltpu.CompilerParams(
            use_tc_tiling_on_sc=True, needs_layout_passes=False),
    )
    return run(sidx, idx_flat, table2d, tails)


def kernel(inputs, tables):
    tt = jnp.transpose(tables, (0, 2, 1))
    table2d = tt.reshape(_F * _DIM, _VOCAB)

    f_ids = jnp.arange(_F, dtype=jnp.int32)[None, :] * _DIM
    d_ids = jnp.arange(_DIM, dtype=jnp.int32)[:, None]
    sidx = jnp.pad((f_ids + d_ids)[:, :, None],
                   ((0, 0), (0, 0), (0, 7))).reshape(-1)

    idx_flat = jnp.transpose(inputs.astype(jnp.int32), (1, 0)).reshape(-1)
    tails = tt[:, :, _VMAIN:].reshape(-1)
    out_flat = _dim_gather_sum(sidx, idx_flat, table2d, tails)
    return jnp.transpose(out_flat.reshape(_DIM, _B), (1, 0))
